# baseline (speedup 1.0000x reference)
import functools

import jax
import jax.numpy as jnp
from jax import lax
from jax.experimental import pallas as pl
from jax.experimental.pallas import tpu as pltpu
from jax.experimental.pallas import tpu_sc as plsc

N = 10000
E = 320000
D = 128

NC = 2
NS = 16
NW = NC * NS
EPW = E // NW
CHUNK = 80
NCHUNK = EPW // CHUNK
NBLK = 5
BCHUNK = NCHUNK // NBLK
NPAD = 10240
STRIPE = NPAD // NS
ZROWS = STRIPE // 4

_MESH = plsc.VectorSubcoreMesh(core_axis_name="c", subcore_axis_name="s")


@functools.partial(
    pl.kernel,
    out_type=jax.ShapeDtypeStruct((NC, 2, NPAD), jnp.float32),
    mesh=_MESH,
    scratch_types=[
        pltpu.VMEM((NCHUNK, CHUNK), jnp.int32),
        pltpu.VMEM((NCHUNK, CHUNK), jnp.int32),
        pltpu.VMEM((CHUNK,), jnp.float32),
        pltpu.VMEM_SHARED((NPAD,), jnp.float32),
        pltpu.VMEM_SHARED((NPAD,), jnp.float32),
    ],
)
def _sc_degrees(src_hbm, dst_hbm, ones_hbm, zeros1_hbm, out_hbm,
                src_v, dst_v, ones_v, acc_out, acc_in):
    cid = lax.axis_index("c")
    sid = lax.axis_index("s")
    wid = sid * NC + cid
    base = sid * STRIPE
    pltpu.sync_copy(zeros1_hbm, acc_out.at[pl.ds(base, STRIPE)])
    pltpu.sync_copy(zeros1_hbm, acc_in.at[pl.ds(base, STRIPE)])
    pltpu.sync_copy(src_hbm.at[wid], src_v)
    pltpu.sync_copy(dst_hbm.at[wid], dst_v)
    pltpu.sync_copy(ones_hbm, ones_v)
    plsc.subcore_barrier()

    def body(j, carry):
        pltpu.sync_copy(ones_v, acc_out.at[src_v.at[j]], add=True)
        pltpu.sync_copy(ones_v, acc_in.at[dst_v.at[j]], add=True)
        return carry

    lax.fori_loop(0, NCHUNK, body, 0)
    plsc.subcore_barrier()
    pltpu.sync_copy(acc_out.at[pl.ds(base, STRIPE)],
                    out_hbm.at[cid, 0, pl.ds(base, STRIPE)])
    pltpu.sync_copy(acc_in.at[pl.ds(base, STRIPE)],
                    out_hbm.at[cid, 1, pl.ds(base, STRIPE)])


@functools.partial(
    pl.kernel,
    out_type=jax.ShapeDtypeStruct((NC, NPAD, D), jnp.float32),
    mesh=_MESH,
    scratch_types=[
        pltpu.VMEM((BCHUNK, CHUNK), jnp.int32),
        pltpu.VMEM((BCHUNK, CHUNK), jnp.int32),
        pltpu.VMEM((2, CHUNK, D), jnp.float32),
        pltpu.VMEM_SHARED((NPAD, D), jnp.float32),
        pltpu.SemaphoreType.DMA,
        pltpu.SemaphoreType.DMA,
        pltpu.SemaphoreType.DMA,
        pltpu.SemaphoreType.DMA,
    ],
)
def _sc_scatter_rows(src_hbm, dst_hbm, xs_hbm, zeros2_hbm, out_hbm,
                     src_v, dst_v, rows_v, acc, sem0, sem1, ssem0, ssem1):
    cid = lax.axis_index("c")
    sid = lax.axis_index("s")
    wid = sid * NC + cid
    base = sid * STRIPE

    def zbody(i, carry):
        pltpu.async_copy(zeros2_hbm, acc.at[pl.ds(base + i * ZROWS, ZROWS)],
                         sem0)
        return carry

    lax.fori_loop(0, 4, zbody, 0)

    def zdrain(i, carry):
        pltpu.make_async_copy(zeros2_hbm,
                              acc.at[pl.ds(base + i * ZROWS, ZROWS)],
                              sem0).wait()
        return carry

    lax.fori_loop(0, 4, zdrain, 0)
    plsc.subcore_barrier()

    def gstart(c, b, sem):
        pltpu.async_copy(xs_hbm.at[src_v.at[c]], rows_v.at[b], sem)

    def gwait(c, b, sem):
        pltpu.make_async_copy(xs_hbm.at[src_v.at[c]],
                              rows_v.at[b], sem).wait()

    def sstart(c, b, sem):
        pltpu.async_copy(rows_v.at[b], acc.at[dst_v.at[c]], sem, add=True)

    def swait(c, b, sem):
        pltpu.make_async_copy(rows_v.at[b], acc.at[dst_v.at[c]],
                              sem).wait()

    def blk_body(blk, carry):
        pltpu.sync_copy(src_hbm.at[wid, blk], src_v)
        pltpu.sync_copy(dst_hbm.at[wid, blk], dst_v)
        gstart(0, 0, sem0)
        gwait(0, 0, sem0)
        sstart(0, 0, ssem0)
        gstart(1, 1, sem1)

        def body(k, c):
            j = 2 * k + 1
            gwait(j, 1, sem1)
            sstart(j, 1, ssem1)
            swait(j - 1, 0, ssem0)
            gstart(j + 1, 0, sem0)
            gwait(j + 1, 0, sem0)
            sstart(j + 1, 0, ssem0)
            swait(j, 1, ssem1)
            gstart(j + 2, 1, sem1)
            return c

        lax.fori_loop(0, (BCHUNK - 3) // 2, body, 0)
        gwait(BCHUNK - 2, 1, sem1)
        sstart(BCHUNK - 2, 1, ssem1)
        swait(BCHUNK - 3, 0, ssem0)
        gstart(BCHUNK - 1, 0, sem0)
        gwait(BCHUNK - 1, 0, sem0)
        sstart(BCHUNK - 1, 0, ssem0)
        swait(BCHUNK - 2, 1, ssem1)
        swait(BCHUNK - 1, 0, ssem0)
        return carry

    lax.fori_loop(0, NBLK, blk_body, 0)
    plsc.subcore_barrier()
    pltpu.sync_copy(acc.at[pl.ds(base, STRIPE)],
                    out_hbm.at[cid, pl.ds(base, STRIPE)])


def _tc_norms_body(deg_ref, h_ref, xs_ref, nout_ref, nin_ref):
    out_deg = deg_ref[0, 0] + deg_ref[1, 0]
    in_deg = deg_ref[0, 1] + deg_ref[1, 1]
    norm_out = lax.rsqrt(jnp.maximum(out_deg, 1.0))[:N]
    norm_in = lax.rsqrt(jnp.maximum(in_deg, 1.0))[:N]
    xs_ref[...] = h_ref[...] * norm_out
    nout_ref[...] = norm_out
    nin_ref[...] = norm_in


def _tc_layer1_body(aggp_ref, nin_ref, nout_ref, w_ref, b_ref,
                    xs2_ref, skip_ref):
    agg = (aggp_ref[0] + aggp_ref[1])[:N] * nin_ref[...]
    x = jnp.dot(agg, w_ref[...], preferred_element_type=jnp.float32)
    x = jnp.maximum(x + b_ref[...], 0.0)
    skip_ref[...] = jnp.sum(x, axis=0, keepdims=True) * (1.0 / N)
    xs2_ref[...] = x * nout_ref[...]


def _tc_layer2_body(aggp_ref, nin_ref, w_ref, b_ref, skip1_ref, out_ref):
    agg = (aggp_ref[0] + aggp_ref[1])[:N] * nin_ref[...]
    x = jnp.dot(agg, w_ref[...], preferred_element_type=jnp.float32)
    x = jnp.maximum(x + b_ref[...], 0.0)
    out_ref[...] = skip1_ref[...] + 2.0 * (jnp.sum(x, axis=0, keepdims=True)
                                           * (1.0 / N))


_tc_norms = pl.pallas_call(
    _tc_norms_body,
    out_shape=(
        jax.ShapeDtypeStruct((N, D), jnp.float32),
        jax.ShapeDtypeStruct((N, 1), jnp.float32),
        jax.ShapeDtypeStruct((N, 1), jnp.float32),
    ),
)

_tc_layer1 = pl.pallas_call(
    _tc_layer1_body,
    out_shape=(
        jax.ShapeDtypeStruct((N, D), jnp.float32),
        jax.ShapeDtypeStruct((1, D), jnp.float32),
    ),
)

_tc_layer2 = pl.pallas_call(
    _tc_layer2_body,
    out_shape=jax.ShapeDtypeStruct((1, D), jnp.float32),
)


@jax.jit
def kernel(h, edge_index, W1, b1, W2, b2):
    src3 = edge_index[0].reshape(NW, NCHUNK, CHUNK)
    dst3 = edge_index[1].reshape(NW, NCHUNK, CHUNK)
    src4 = src3.reshape(NW, NBLK, BCHUNK, CHUNK)
    dst4 = dst3.reshape(NW, NBLK, BCHUNK, CHUNK)
    ones = jnp.ones((CHUNK,), jnp.float32)
    zeros1 = jnp.zeros((STRIPE,), jnp.float32)
    zeros2 = jnp.zeros((ZROWS, D), jnp.float32)

    deg = _sc_degrees(src3, dst3, ones, zeros1)
    deg4 = deg.reshape(NC, 2, NPAD, 1)
    xs1, norm_out, norm_in = _tc_norms(deg4, h)

    agg1 = _sc_scatter_rows(src4, dst4, xs1, zeros2)
    xs2, skip1 = _tc_layer1(agg1, norm_in, norm_out, W1, b1.reshape(1, D))

    agg2 = _sc_scatter_rows(src4, dst4, xs2, zeros2)
    return _tc_layer2(agg2, norm_in, W2, b2.reshape(1, D), skip1)


# speedup vs baseline: 1.2988x; 1.2988x over previous
import functools

import jax
import jax.numpy as jnp
from jax import lax
from jax.experimental import pallas as pl
from jax.experimental.pallas import tpu as pltpu
from jax.experimental.pallas import tpu_sc as plsc

N = 10000
E = 320000
D = 128

NC = 2
NS = 16
NW = NC * NS
EPW = E // NW
CHUNK = 80
NCHUNK = EPW // CHUNK
NBLK = 5
BCHUNK = NCHUNK // NBLK
NPAD = 10240
STRIPE = NPAD // NS
ZROWS = STRIPE // 4

_MESH = plsc.VectorSubcoreMesh(core_axis_name="c", subcore_axis_name="s")


@functools.partial(
    pl.kernel,
    out_type=jax.ShapeDtypeStruct((NC, 2, NPAD), jnp.float32),
    mesh=_MESH,
    scratch_types=[
        pltpu.VMEM((NCHUNK, CHUNK), jnp.int32),
        pltpu.VMEM((NCHUNK, CHUNK), jnp.int32),
        pltpu.VMEM((CHUNK,), jnp.float32),
        pltpu.VMEM_SHARED((NPAD,), jnp.float32),
        pltpu.VMEM_SHARED((NPAD,), jnp.float32),
    ],
)
def _sc_degrees(src_hbm, dst_hbm, ones_hbm, zeros1_hbm, out_hbm,
                src_v, dst_v, ones_v, acc_out, acc_in):
    cid = lax.axis_index("c")
    sid = lax.axis_index("s")
    wid = sid * NC + cid
    base = sid * STRIPE
    pltpu.sync_copy(zeros1_hbm, acc_out.at[pl.ds(base, STRIPE)])
    pltpu.sync_copy(zeros1_hbm, acc_in.at[pl.ds(base, STRIPE)])
    pltpu.sync_copy(src_hbm.at[wid], src_v)
    pltpu.sync_copy(dst_hbm.at[wid], dst_v)
    pltpu.sync_copy(ones_hbm, ones_v)
    plsc.subcore_barrier()

    def body(j, carry):
        pltpu.sync_copy(ones_v, acc_out.at[src_v.at[j]], add=True)
        pltpu.sync_copy(ones_v, acc_in.at[dst_v.at[j]], add=True)
        return carry

    lax.fori_loop(0, NCHUNK, body, 0)
    plsc.subcore_barrier()
    pltpu.sync_copy(acc_out.at[pl.ds(base, STRIPE)],
                    out_hbm.at[cid, 0, pl.ds(base, STRIPE)])
    pltpu.sync_copy(acc_in.at[pl.ds(base, STRIPE)],
                    out_hbm.at[cid, 1, pl.ds(base, STRIPE)])


@functools.partial(
    pl.kernel,
    out_type=jax.ShapeDtypeStruct((NC, NPAD, D), jnp.float32),
    mesh=_MESH,
    scratch_types=[
        pltpu.VMEM((BCHUNK, CHUNK), jnp.int32),
        pltpu.VMEM((BCHUNK, CHUNK), jnp.int32),
        pltpu.VMEM((3, CHUNK, D), jnp.float32),
        pltpu.VMEM_SHARED((NPAD, D), jnp.float32),
        pltpu.SemaphoreType.DMA((3,)),
        pltpu.SemaphoreType.DMA,
    ],
)
def _sc_scatter_rows(src_hbm, dst_hbm, xs_hbm, zeros2_hbm, out_hbm,
                     src_v, dst_v, rows_v, acc, sems, zsem):
    cid = lax.axis_index("c")
    sid = lax.axis_index("s")
    wid = sid * NC + cid
    base = sid * STRIPE

    def zbody(i, carry):
        pltpu.async_copy(zeros2_hbm, acc.at[pl.ds(base + i * ZROWS, ZROWS)],
                         zsem)
        return carry

    lax.fori_loop(0, 4, zbody, 0)

    def zdrain(i, carry):
        pltpu.make_async_copy(zeros2_hbm,
                              acc.at[pl.ds(base + i * ZROWS, ZROWS)],
                              zsem).wait()
        return carry

    lax.fori_loop(0, 4, zdrain, 0)
    plsc.subcore_barrier()

    def gstart(c, b):
        pltpu.async_copy(xs_hbm.at[src_v.at[c]], rows_v.at[b], sems.at[b])

    def gwait(c, b):
        pltpu.make_async_copy(xs_hbm.at[src_v.at[c]],
                              rows_v.at[b], sems.at[b]).wait()

    def sstart(c, b):
        pltpu.async_copy(rows_v.at[b], acc.at[dst_v.at[c]], sems.at[b],
                         add=True)

    def swait(c, b):
        pltpu.make_async_copy(rows_v.at[b], acc.at[dst_v.at[c]],
                              sems.at[b]).wait()

    def blk_body(blk, carry):
        pltpu.sync_copy(src_hbm.at[wid, blk], src_v)
        pltpu.sync_copy(dst_hbm.at[wid, blk], dst_v)
        gstart(0, 0)
        gstart(1, 1)

        def body(s, c):
            b = lax.rem(s, 3)
            gwait(s, b)
            sstart(s, b)

            @pl.when(jnp.logical_and(s >= 1, s + 2 < BCHUNK))
            def _():
                swait(s - 1, lax.rem(s - 1, 3))

            @pl.when(s + 2 < BCHUNK)
            def _():
                gstart(s + 2, lax.rem(s + 2, 3))

            return c

        lax.fori_loop(0, BCHUNK, body, 0)
        swait(BCHUNK - 3, (BCHUNK - 3) % 3)
        swait(BCHUNK - 2, (BCHUNK - 2) % 3)
        swait(BCHUNK - 1, (BCHUNK - 1) % 3)
        return carry

    lax.fori_loop(0, NBLK, blk_body, 0)
    plsc.subcore_barrier()
    pltpu.sync_copy(acc.at[pl.ds(base, STRIPE)],
                    out_hbm.at[cid, pl.ds(base, STRIPE)])


def _tc_norms_body(deg_ref, h_ref, xs_ref, nout_ref, nin_ref):
    out_deg = deg_ref[0, 0] + deg_ref[1, 0]
    in_deg = deg_ref[0, 1] + deg_ref[1, 1]
    norm_out = lax.rsqrt(jnp.maximum(out_deg, 1.0))[:N]
    norm_in = lax.rsqrt(jnp.maximum(in_deg, 1.0))[:N]
    xs_ref[...] = h_ref[...] * norm_out
    nout_ref[...] = norm_out
    nin_ref[...] = norm_in


def _tc_layer1_body(aggp_ref, nin_ref, nout_ref, w_ref, b_ref,
                    xs2_ref, skip_ref):
    agg = (aggp_ref[0] + aggp_ref[1])[:N] * nin_ref[...]
    x = jnp.dot(agg, w_ref[...], preferred_element_type=jnp.float32)
    x = jnp.maximum(x + b_ref[...], 0.0)
    skip_ref[...] = jnp.sum(x, axis=0, keepdims=True) * (1.0 / N)
    xs2_ref[...] = x * nout_ref[...]


def _tc_layer2_body(aggp_ref, nin_ref, w_ref, b_ref, skip1_ref, out_ref):
    agg = (aggp_ref[0] + aggp_ref[1])[:N] * nin_ref[...]
    x = jnp.dot(agg, w_ref[...], preferred_element_type=jnp.float32)
    x = jnp.maximum(x + b_ref[...], 0.0)
    out_ref[...] = skip1_ref[...] + 2.0 * (jnp.sum(x, axis=0, keepdims=True)
                                           * (1.0 / N))


_tc_norms = pl.pallas_call(
    _tc_norms_body,
    out_shape=(
        jax.ShapeDtypeStruct((N, D), jnp.float32),
        jax.ShapeDtypeStruct((N, 1), jnp.float32),
        jax.ShapeDtypeStruct((N, 1), jnp.float32),
    ),
)

_tc_layer1 = pl.pallas_call(
    _tc_layer1_body,
    out_shape=(
        jax.ShapeDtypeStruct((N, D), jnp.float32),
        jax.ShapeDtypeStruct((1, D), jnp.float32),
    ),
)

_tc_layer2 = pl.pallas_call(
    _tc_layer2_body,
    out_shape=jax.ShapeDtypeStruct((1, D), jnp.float32),
)


@jax.jit
def kernel(h, edge_index, W1, b1, W2, b2):
    src3 = edge_index[0].reshape(NW, NCHUNK, CHUNK)
    dst3 = edge_index[1].reshape(NW, NCHUNK, CHUNK)
    src4 = src3.reshape(NW, NBLK, BCHUNK, CHUNK)
    dst4 = dst3.reshape(NW, NBLK, BCHUNK, CHUNK)
    ones = jnp.ones((CHUNK,), jnp.float32)
    zeros1 = jnp.zeros((STRIPE,), jnp.float32)
    zeros2 = jnp.zeros((ZROWS, D), jnp.float32)

    deg = _sc_degrees(src3, dst3, ones, zeros1)
    deg4 = deg.reshape(NC, 2, NPAD, 1)
    xs1, norm_out, norm_in = _tc_norms(deg4, h)

    agg1 = _sc_scatter_rows(src4, dst4, xs1, zeros2)
    xs2, skip1 = _tc_layer1(agg1, norm_in, norm_out, W1, b1.reshape(1, D))

    agg2 = _sc_scatter_rows(src4, dst4, xs2, zeros2)
    return _tc_layer2(agg2, norm_in, W2, b2.reshape(1, D), skip1)


# speedup vs baseline: 1.3719x; 1.0562x over previous
import functools

import jax
import jax.numpy as jnp
from jax import lax
from jax.experimental import pallas as pl
from jax.experimental.pallas import tpu as pltpu
from jax.experimental.pallas import tpu_sc as plsc

N = 10000
E = 320000
D = 128

NC = 2
NS = 16
NW = NC * NS
EPW = E // NW
CHUNK = 80
NCHUNK = EPW // CHUNK
NBLK = 5
BCHUNK = NCHUNK // NBLK
NPAD = 10240
STRIPE = NPAD // NS
ZROWS = STRIPE // 4

_MESH = plsc.VectorSubcoreMesh(core_axis_name="c", subcore_axis_name="s")


@functools.partial(
    pl.kernel,
    out_type=jax.ShapeDtypeStruct((NC, 2, NPAD), jnp.float32),
    mesh=_MESH,
    scratch_types=[
        pltpu.VMEM((NCHUNK, CHUNK), jnp.int32),
        pltpu.VMEM((NCHUNK, CHUNK), jnp.int32),
        pltpu.VMEM((CHUNK,), jnp.float32),
        pltpu.VMEM_SHARED((NPAD,), jnp.float32),
        pltpu.VMEM_SHARED((NPAD,), jnp.float32),
    ],
)
def _sc_degrees(src_hbm, dst_hbm, ones_hbm, zeros1_hbm, out_hbm,
                src_v, dst_v, ones_v, acc_out, acc_in):
    cid = lax.axis_index("c")
    sid = lax.axis_index("s")
    wid = sid * NC + cid
    base = sid * STRIPE
    pltpu.sync_copy(zeros1_hbm, acc_out.at[pl.ds(base, STRIPE)])
    pltpu.sync_copy(zeros1_hbm, acc_in.at[pl.ds(base, STRIPE)])
    pltpu.sync_copy(src_hbm.at[wid], src_v)
    pltpu.sync_copy(dst_hbm.at[wid], dst_v)
    pltpu.sync_copy(ones_hbm, ones_v)
    plsc.subcore_barrier()

    def body(j, carry):
        pltpu.sync_copy(ones_v, acc_out.at[src_v.at[j]], add=True)
        pltpu.sync_copy(ones_v, acc_in.at[dst_v.at[j]], add=True)
        return carry

    lax.fori_loop(0, NCHUNK, body, 0)
    plsc.subcore_barrier()
    pltpu.sync_copy(acc_out.at[pl.ds(base, STRIPE)],
                    out_hbm.at[cid, 0, pl.ds(base, STRIPE)])
    pltpu.sync_copy(acc_in.at[pl.ds(base, STRIPE)],
                    out_hbm.at[cid, 1, pl.ds(base, STRIPE)])


@functools.partial(
    pl.kernel,
    out_type=jax.ShapeDtypeStruct((NC, NPAD, D), jnp.float32),
    mesh=_MESH,
    scratch_types=[
        pltpu.VMEM((2, BCHUNK, CHUNK), jnp.int32),
        pltpu.VMEM((2, BCHUNK, CHUNK), jnp.int32),
        pltpu.VMEM((3, CHUNK, D), jnp.float32),
        pltpu.VMEM_SHARED((NPAD, D), jnp.float32),
        pltpu.SemaphoreType.DMA((3,)),
        pltpu.SemaphoreType.DMA,
    ],
)
def _sc_scatter_rows(src_hbm, dst_hbm, xs_hbm, zeros2_hbm, out_hbm,
                     src_v, dst_v, rows_v, acc, sems, zsem):
    cid = lax.axis_index("c")
    sid = lax.axis_index("s")
    wid = sid * NC + cid
    base = sid * STRIPE

    def zbody(i, carry):
        pltpu.async_copy(zeros2_hbm, acc.at[pl.ds(base + i * ZROWS, ZROWS)],
                         zsem)
        return carry

    lax.fori_loop(0, 4, zbody, 0)

    def zdrain(i, carry):
        pltpu.make_async_copy(zeros2_hbm,
                              acc.at[pl.ds(base + i * ZROWS, ZROWS)],
                              zsem).wait()
        return carry

    lax.fori_loop(0, 4, zdrain, 0)
    plsc.subcore_barrier()

    def gstart(p, o, b):
        pltpu.async_copy(xs_hbm.at[src_v.at[p, o]], rows_v.at[b],
                         sems.at[b])

    def gwait(p, o, b):
        pltpu.make_async_copy(xs_hbm.at[src_v.at[p, o]],
                              rows_v.at[b], sems.at[b]).wait()

    def sstart(p, o, b):
        pltpu.async_copy(rows_v.at[b], acc.at[dst_v.at[p, o]], sems.at[b],
                         add=True)

    def swait(p, o, b):
        pltpu.make_async_copy(rows_v.at[b], acc.at[dst_v.at[p, o]],
                              sems.at[b]).wait()

    pltpu.sync_copy(src_hbm.at[wid, 0], src_v.at[0])
    pltpu.sync_copy(dst_hbm.at[wid, 0], dst_v.at[0])
    gstart(0, 0, 0)
    gstart(0, 1, 1)

    def blk_body(blk, carry):
        p = lax.rem(blk, 2)

        def body(o, c_):
            c = blk * BCHUNK + o
            b = lax.rem(c, 3)

            gwait(p, o, b)
            sstart(p, o, b)

            @pl.when(c >= 1)
            def _():
                swait(p, 0, lax.rem(c + 2, 3))

            @pl.when(jnp.logical_and(o == 0, blk + 1 < NBLK))
            def _():
                pltpu.async_copy(src_hbm.at[wid, blk + 1],
                                 src_v.at[1 - p], zsem)
                pltpu.async_copy(dst_hbm.at[wid, blk + 1],
                                 dst_v.at[1 - p], zsem)

            @pl.when(jnp.logical_and(o == BCHUNK - 2, blk + 1 < NBLK))
            def _():
                pltpu.make_async_copy(src_hbm.at[wid, blk + 1],
                                      src_v.at[1 - p], zsem).wait()
                pltpu.make_async_copy(dst_hbm.at[wid, blk + 1],
                                      dst_v.at[1 - p], zsem).wait()

            @pl.when(c + 2 < NCHUNK)
            def _():
                nxt = o >= BCHUNK - 2
                p2 = lax.rem(blk + nxt.astype(jnp.int32), 2)
                o2 = jnp.where(nxt, o + 2 - BCHUNK, o + 2)
                gstart(p2, o2, lax.rem(c + 2, 3))

            return c_

        lax.fori_loop(0, BCHUNK, body, 0)
        return carry

    lax.fori_loop(0, NBLK, blk_body, 0)
    swait((NBLK - 1) % 2, 0, (NCHUNK - 1) % 3)
    plsc.subcore_barrier()
    pltpu.sync_copy(acc.at[pl.ds(base, STRIPE)],
                    out_hbm.at[cid, pl.ds(base, STRIPE)])


def _tc_norms_body(deg_ref, h_ref, xs_ref, nout_ref, nin_ref):
    out_deg = deg_ref[0, 0] + deg_ref[1, 0]
    in_deg = deg_ref[0, 1] + deg_ref[1, 1]
    norm_out = lax.rsqrt(jnp.maximum(out_deg, 1.0))[:N]
    norm_in = lax.rsqrt(jnp.maximum(in_deg, 1.0))[:N]
    xs_ref[...] = h_ref[...] * norm_out
    nout_ref[...] = norm_out
    nin_ref[...] = norm_in


def _tc_layer1_body(aggp_ref, nin_ref, nout_ref, w_ref, b_ref,
                    xs2_ref, skip_ref):
    agg = (aggp_ref[0] + aggp_ref[1])[:N] * nin_ref[...]
    x = jnp.dot(agg, w_ref[...], preferred_element_type=jnp.float32)
    x = jnp.maximum(x + b_ref[...], 0.0)
    skip_ref[...] = jnp.sum(x, axis=0, keepdims=True) * (1.0 / N)
    xs2_ref[...] = x * nout_ref[...]


def _tc_layer2_body(aggp_ref, nin_ref, w_ref, b_ref, skip1_ref, out_ref):
    agg = (aggp_ref[0] + aggp_ref[1])[:N] * nin_ref[...]
    x = jnp.dot(agg, w_ref[...], preferred_element_type=jnp.float32)
    x = jnp.maximum(x + b_ref[...], 0.0)
    out_ref[...] = skip1_ref[...] + 2.0 * (jnp.sum(x, axis=0, keepdims=True)
                                           * (1.0 / N))


_tc_norms = pl.pallas_call(
    _tc_norms_body,
    out_shape=(
        jax.ShapeDtypeStruct((N, D), jnp.float32),
        jax.ShapeDtypeStruct((N, 1), jnp.float32),
        jax.ShapeDtypeStruct((N, 1), jnp.float32),
    ),
)

_tc_layer1 = pl.pallas_call(
    _tc_layer1_body,
    out_shape=(
        jax.ShapeDtypeStruct((N, D), jnp.float32),
        jax.ShapeDtypeStruct((1, D), jnp.float32),
    ),
)

_tc_layer2 = pl.pallas_call(
    _tc_layer2_body,
    out_shape=jax.ShapeDtypeStruct((1, D), jnp.float32),
)


@jax.jit
def kernel(h, edge_index, W1, b1, W2, b2):
    src3 = edge_index[0].reshape(NW, NCHUNK, CHUNK)
    dst3 = edge_index[1].reshape(NW, NCHUNK, CHUNK)
    src4 = src3.reshape(NW, NBLK, BCHUNK, CHUNK)
    dst4 = dst3.reshape(NW, NBLK, BCHUNK, CHUNK)
    ones = jnp.ones((CHUNK,), jnp.float32)
    zeros1 = jnp.zeros((STRIPE,), jnp.float32)
    zeros2 = jnp.zeros((ZROWS, D), jnp.float32)

    deg = _sc_degrees(src3, dst3, ones, zeros1)
    deg4 = deg.reshape(NC, 2, NPAD, 1)
    xs1, norm_out, norm_in = _tc_norms(deg4, h)

    agg1 = _sc_scatter_rows(src4, dst4, xs1, zeros2)
    xs2, skip1 = _tc_layer1(agg1, norm_in, norm_out, W1, b1.reshape(1, D))

    agg2 = _sc_scatter_rows(src4, dst4, xs2, zeros2)
    return _tc_layer2(agg2, norm_in, W2, b2.reshape(1, D), skip1)


# speedup vs baseline: 1.4240x; 1.0380x over previous
import functools

import jax
import jax.numpy as jnp
from jax import lax
from jax.experimental import pallas as pl
from jax.experimental.pallas import tpu as pltpu
from jax.experimental.pallas import tpu_sc as plsc

N = 10000
E = 320000
D = 128

NC = 2
NS = 16
NW = NC * NS
EPW = E // NW
CHUNK = 80
NCHUNK = EPW // CHUNK
NBLK = 5
BCHUNK = NCHUNK // NBLK
NPAD = 10240
STRIPE = NPAD // NS
ZROWS = STRIPE // 4

_MESH = plsc.VectorSubcoreMesh(core_axis_name="c", subcore_axis_name="s")


@functools.partial(
    pl.kernel,
    out_type=jax.ShapeDtypeStruct((NC, 2, NPAD), jnp.float32),
    mesh=_MESH,
    scratch_types=[
        pltpu.VMEM((NCHUNK, CHUNK), jnp.int32),
        pltpu.VMEM((NCHUNK, CHUNK), jnp.int32),
        pltpu.VMEM((CHUNK,), jnp.float32),
        pltpu.VMEM_SHARED((NPAD,), jnp.float32),
        pltpu.VMEM_SHARED((NPAD,), jnp.float32),
        pltpu.SemaphoreType.DMA((2,)),
    ],
)
def _sc_degrees(src_hbm, dst_hbm, ones_hbm, zeros1_hbm, out_hbm,
                src_v, dst_v, ones_v, acc_out, acc_in, sems):
    cid = lax.axis_index("c")
    sid = lax.axis_index("s")
    wid = sid * NC + cid
    base = sid * STRIPE
    pltpu.sync_copy(zeros1_hbm, acc_out.at[pl.ds(base, STRIPE)])
    pltpu.sync_copy(zeros1_hbm, acc_in.at[pl.ds(base, STRIPE)])
    pltpu.sync_copy(src_hbm.at[wid], src_v)
    pltpu.sync_copy(dst_hbm.at[wid], dst_v)
    pltpu.sync_copy(ones_hbm, ones_v)
    plsc.subcore_barrier()

    def body(j, carry):
        pltpu.async_copy(ones_v, acc_out.at[src_v.at[j]], sems.at[0],
                         add=True)
        pltpu.async_copy(ones_v, acc_in.at[dst_v.at[j]], sems.at[1],
                         add=True)

        @pl.when(j >= 1)
        def _():
            pltpu.make_async_copy(ones_v, acc_out.at[src_v.at[0]],
                                  sems.at[0]).wait()
            pltpu.make_async_copy(ones_v, acc_in.at[dst_v.at[0]],
                                  sems.at[1]).wait()

        return carry

    lax.fori_loop(0, NCHUNK, body, 0)
    pltpu.make_async_copy(ones_v, acc_out.at[src_v.at[0]],
                          sems.at[0]).wait()
    pltpu.make_async_copy(ones_v, acc_in.at[dst_v.at[0]],
                          sems.at[1]).wait()
    plsc.subcore_barrier()
    pltpu.sync_copy(acc_out.at[pl.ds(base, STRIPE)],
                    out_hbm.at[cid, 0, pl.ds(base, STRIPE)])
    pltpu.sync_copy(acc_in.at[pl.ds(base, STRIPE)],
                    out_hbm.at[cid, 1, pl.ds(base, STRIPE)])


@functools.partial(
    pl.kernel,
    out_type=jax.ShapeDtypeStruct((NC, NPAD, D), jnp.float32),
    mesh=_MESH,
    scratch_types=[
        pltpu.VMEM((2, BCHUNK, CHUNK), jnp.int32),
        pltpu.VMEM((2, BCHUNK, CHUNK), jnp.int32),
        pltpu.VMEM((3, CHUNK, D), jnp.float32),
        pltpu.VMEM_SHARED((NPAD, D), jnp.float32),
        pltpu.SemaphoreType.DMA((3,)),
        pltpu.SemaphoreType.DMA,
    ],
)
def _sc_scatter_rows(src_hbm, dst_hbm, xs_hbm, zeros2_hbm, out_hbm,
                     src_v, dst_v, rows_v, acc, sems, zsem):
    cid = lax.axis_index("c")
    sid = lax.axis_index("s")
    wid = sid * NC + cid
    base = sid * STRIPE

    def zbody(i, carry):
        pltpu.async_copy(zeros2_hbm, acc.at[pl.ds(base + i * ZROWS, ZROWS)],
                         zsem)
        return carry

    lax.fori_loop(0, 4, zbody, 0)

    def zdrain(i, carry):
        pltpu.make_async_copy(zeros2_hbm,
                              acc.at[pl.ds(base + i * ZROWS, ZROWS)],
                              zsem).wait()
        return carry

    lax.fori_loop(0, 4, zdrain, 0)
    plsc.subcore_barrier()

    def gstart(p, o, b):
        pltpu.async_copy(xs_hbm.at[src_v.at[p, o]], rows_v.at[b],
                         sems.at[b])

    def gwait(p, o, b):
        pltpu.make_async_copy(xs_hbm.at[src_v.at[p, o]],
                              rows_v.at[b], sems.at[b]).wait()

    def sstart(p, o, b):
        pltpu.async_copy(rows_v.at[b], acc.at[dst_v.at[p, o]], sems.at[b],
                         add=True)

    def swait(p, o, b):
        pltpu.make_async_copy(rows_v.at[b], acc.at[dst_v.at[p, o]],
                              sems.at[b]).wait()

    pltpu.sync_copy(src_hbm.at[wid, 0], src_v.at[0])
    pltpu.sync_copy(dst_hbm.at[wid, 0], dst_v.at[0])
    gstart(0, 0, 0)
    gstart(0, 1, 1)

    def blk_body(blk, carry):
        p = lax.rem(blk, 2)

        def body(o, c_):
            c = blk * BCHUNK + o
            b = lax.rem(c, 3)

            gwait(p, o, b)
            sstart(p, o, b)

            @pl.when(c >= 1)
            def _():
                swait(p, 0, lax.rem(c + 2, 3))

            @pl.when(jnp.logical_and(o == 0, blk + 1 < NBLK))
            def _():
                pltpu.async_copy(src_hbm.at[wid, blk + 1],
                                 src_v.at[1 - p], zsem)
                pltpu.async_copy(dst_hbm.at[wid, blk + 1],
                                 dst_v.at[1 - p], zsem)

            @pl.when(jnp.logical_and(o == BCHUNK - 2, blk + 1 < NBLK))
            def _():
                pltpu.make_async_copy(src_hbm.at[wid, blk + 1],
                                      src_v.at[1 - p], zsem).wait()
                pltpu.make_async_copy(dst_hbm.at[wid, blk + 1],
                                      dst_v.at[1 - p], zsem).wait()

            @pl.when(c + 2 < NCHUNK)
            def _():
                nxt = o >= BCHUNK - 2
                p2 = lax.rem(blk + nxt.astype(jnp.int32), 2)
                o2 = jnp.where(nxt, o + 2 - BCHUNK, o + 2)
                gstart(p2, o2, lax.rem(c + 2, 3))

            return c_

        lax.fori_loop(0, BCHUNK, body, 0)
        return carry

    lax.fori_loop(0, NBLK, blk_body, 0)
    swait((NBLK - 1) % 2, 0, (NCHUNK - 1) % 3)
    plsc.subcore_barrier()
    pltpu.sync_copy(acc.at[pl.ds(base, STRIPE)],
                    out_hbm.at[cid, pl.ds(base, STRIPE)])


def _tc_norms_body(deg_ref, h_ref, xs_ref, nout_ref, nin_ref):
    out_deg = deg_ref[0, 0] + deg_ref[1, 0]
    in_deg = deg_ref[0, 1] + deg_ref[1, 1]
    norm_out = lax.rsqrt(jnp.maximum(out_deg, 1.0))[:N]
    norm_in = lax.rsqrt(jnp.maximum(in_deg, 1.0))[:N]
    xs_ref[...] = h_ref[...] * norm_out
    nout_ref[...] = norm_out
    nin_ref[...] = norm_in


def _tc_layer1_body(aggp_ref, nin_ref, nout_ref, w_ref, b_ref,
                    xs2_ref, skip_ref):
    agg = (aggp_ref[0] + aggp_ref[1])[:N] * nin_ref[...]
    x = jnp.dot(agg, w_ref[...], preferred_element_type=jnp.float32)
    x = jnp.maximum(x + b_ref[...], 0.0)
    skip_ref[...] = jnp.sum(x, axis=0, keepdims=True) * (1.0 / N)
    xs2_ref[...] = x * nout_ref[...]


def _tc_layer2_body(aggp_ref, nin_ref, w_ref, b_ref, skip1_ref, out_ref):
    agg = (aggp_ref[0] + aggp_ref[1])[:N] * nin_ref[...]
    x = jnp.dot(agg, w_ref[...], preferred_element_type=jnp.float32)
    x = jnp.maximum(x + b_ref[...], 0.0)
    out_ref[...] = skip1_ref[...] + 2.0 * (jnp.sum(x, axis=0, keepdims=True)
                                           * (1.0 / N))


_tc_norms = pl.pallas_call(
    _tc_norms_body,
    out_shape=(
        jax.ShapeDtypeStruct((N, D), jnp.float32),
        jax.ShapeDtypeStruct((N, 1), jnp.float32),
        jax.ShapeDtypeStruct((N, 1), jnp.float32),
    ),
)

_tc_layer1 = pl.pallas_call(
    _tc_layer1_body,
    out_shape=(
        jax.ShapeDtypeStruct((N, D), jnp.float32),
        jax.ShapeDtypeStruct((1, D), jnp.float32),
    ),
)

_tc_layer2 = pl.pallas_call(
    _tc_layer2_body,
    out_shape=jax.ShapeDtypeStruct((1, D), jnp.float32),
)


@jax.jit
def kernel(h, edge_index, W1, b1, W2, b2):
    src3 = edge_index[0].reshape(NW, NCHUNK, CHUNK)
    dst3 = edge_index[1].reshape(NW, NCHUNK, CHUNK)
    src4 = src3.reshape(NW, NBLK, BCHUNK, CHUNK)
    dst4 = dst3.reshape(NW, NBLK, BCHUNK, CHUNK)
    ones = jnp.ones((CHUNK,), jnp.float32)
    zeros1 = jnp.zeros((STRIPE,), jnp.float32)
    zeros2 = jnp.zeros((ZROWS, D), jnp.float32)

    deg = _sc_degrees(src3, dst3, ones, zeros1)
    deg4 = deg.reshape(NC, 2, NPAD, 1)
    xs1, norm_out, norm_in = _tc_norms(deg4, h)

    agg1 = _sc_scatter_rows(src4, dst4, xs1, zeros2)
    xs2, skip1 = _tc_layer1(agg1, norm_in, norm_out, W1, b1.reshape(1, D))

    agg2 = _sc_scatter_rows(src4, dst4, xs2, zeros2)
    return _tc_layer2(agg2, norm_in, W2, b2.reshape(1, D), skip1)


# speedup vs baseline: 1.4341x; 1.0071x over previous
import functools

import jax
import jax.numpy as jnp
from jax import lax
from jax.experimental import pallas as pl
from jax.experimental.pallas import tpu as pltpu
from jax.experimental.pallas import tpu_sc as plsc

N = 10000
E = 320000
D = 128

NC = 2
NS = 16
NW = NC * NS
EPW = E // NW
CHUNK = 80
NCHUNK = EPW // CHUNK
NBLK = 5
BCHUNK = NCHUNK // NBLK
NPAD = 10240
STRIPE = NPAD // NS
ZROWS = STRIPE // 4

_MESH = plsc.VectorSubcoreMesh(core_axis_name="c", subcore_axis_name="s")


@functools.partial(
    pl.kernel,
    out_type=jax.ShapeDtypeStruct((NC, 2, NPAD), jnp.float32),
    mesh=_MESH,
    scratch_types=[
        pltpu.VMEM((NCHUNK, CHUNK), jnp.int32),
        pltpu.VMEM((NCHUNK, CHUNK), jnp.int32),
        pltpu.VMEM((CHUNK,), jnp.float32),
        pltpu.VMEM_SHARED((NPAD,), jnp.float32),
        pltpu.VMEM_SHARED((NPAD,), jnp.float32),
        pltpu.SemaphoreType.DMA((2,)),
    ],
)
def _sc_degrees(src_hbm, dst_hbm, ones_hbm, zeros1_hbm, out_hbm,
                src_v, dst_v, ones_v, acc_out, acc_in, sems):
    cid = lax.axis_index("c")
    sid = lax.axis_index("s")
    wid = sid * NC + cid
    base = sid * STRIPE
    pltpu.sync_copy(zeros1_hbm, acc_out.at[pl.ds(base, STRIPE)])
    pltpu.sync_copy(zeros1_hbm, acc_in.at[pl.ds(base, STRIPE)])
    pltpu.sync_copy(src_hbm.at[wid], src_v)
    pltpu.sync_copy(dst_hbm.at[wid], dst_v)
    pltpu.sync_copy(ones_hbm, ones_v)
    plsc.subcore_barrier()

    def body(j, carry):
        pltpu.async_copy(ones_v, acc_out.at[src_v.at[j]], sems.at[0],
                         add=True)
        pltpu.async_copy(ones_v, acc_in.at[dst_v.at[j]], sems.at[1],
                         add=True)

        @pl.when(j >= 1)
        def _():
            pltpu.make_async_copy(ones_v, acc_out.at[src_v.at[0]],
                                  sems.at[0]).wait()
            pltpu.make_async_copy(ones_v, acc_in.at[dst_v.at[0]],
                                  sems.at[1]).wait()

        return carry

    lax.fori_loop(0, NCHUNK, body, 0)
    pltpu.make_async_copy(ones_v, acc_out.at[src_v.at[0]],
                          sems.at[0]).wait()
    pltpu.make_async_copy(ones_v, acc_in.at[dst_v.at[0]],
                          sems.at[1]).wait()
    plsc.subcore_barrier()
    pltpu.sync_copy(acc_out.at[pl.ds(base, STRIPE)],
                    out_hbm.at[cid, 0, pl.ds(base, STRIPE)])
    pltpu.sync_copy(acc_in.at[pl.ds(base, STRIPE)],
                    out_hbm.at[cid, 1, pl.ds(base, STRIPE)])


@functools.partial(
    pl.kernel,
    out_type=jax.ShapeDtypeStruct((NC, NPAD, D), jnp.float32),
    mesh=_MESH,
    scratch_types=[
        pltpu.VMEM((2, BCHUNK, CHUNK), jnp.int32),
        pltpu.VMEM((2, BCHUNK, CHUNK), jnp.int32),
        pltpu.VMEM((3, CHUNK, D), jnp.float32),
        pltpu.VMEM_SHARED((NPAD, D), jnp.float32),
        pltpu.SemaphoreType.DMA((3,)),
        pltpu.SemaphoreType.DMA,
    ],
)
def _sc_scatter_rows(src_hbm, dst_hbm, xs_hbm, zeros2_hbm, out_hbm,
                     src_v, dst_v, rows_v, acc, sems, zsem):
    cid = lax.axis_index("c")
    sid = lax.axis_index("s")
    wid = sid * NC + cid
    base = sid * STRIPE

    def zbody(i, carry):
        pltpu.async_copy(zeros2_hbm, acc.at[pl.ds(base + i * ZROWS, ZROWS)],
                         zsem)
        return carry

    lax.fori_loop(0, 4, zbody, 0)
    pltpu.sync_copy(src_hbm.at[wid, 0], src_v.at[0])
    pltpu.sync_copy(dst_hbm.at[wid, 0], dst_v.at[0])

    def zdrain(i, carry):
        pltpu.make_async_copy(zeros2_hbm,
                              acc.at[pl.ds(base + i * ZROWS, ZROWS)],
                              zsem).wait()
        return carry

    lax.fori_loop(0, 4, zdrain, 0)
    plsc.subcore_barrier()

    def gstart(p, o, b):
        pltpu.async_copy(xs_hbm.at[src_v.at[p, o]], rows_v.at[b],
                         sems.at[b])

    def gwait(p, o, b):
        pltpu.make_async_copy(xs_hbm.at[src_v.at[p, o]],
                              rows_v.at[b], sems.at[b]).wait()

    def sstart(p, o, b):
        pltpu.async_copy(rows_v.at[b], acc.at[dst_v.at[p, o]], sems.at[b],
                         add=True)

    def swait(p, o, b):
        pltpu.make_async_copy(rows_v.at[b], acc.at[dst_v.at[p, o]],
                              sems.at[b]).wait()

    gstart(0, 0, 0)
    gstart(0, 1, 1)

    def blk_body(blk, carry):
        p = lax.rem(blk, 2)

        def body(o, c_):
            c = blk * BCHUNK + o
            b = lax.rem(c, 3)

            gwait(p, o, b)
            sstart(p, o, b)

            @pl.when(c >= 1)
            def _():
                swait(p, 0, lax.rem(c + 2, 3))

            @pl.when(jnp.logical_and(o == 0, blk + 1 < NBLK))
            def _():
                pltpu.async_copy(src_hbm.at[wid, blk + 1],
                                 src_v.at[1 - p], zsem)
                pltpu.async_copy(dst_hbm.at[wid, blk + 1],
                                 dst_v.at[1 - p], zsem)

            @pl.when(jnp.logical_and(o == BCHUNK - 2, blk + 1 < NBLK))
            def _():
                pltpu.make_async_copy(src_hbm.at[wid, blk + 1],
                                      src_v.at[1 - p], zsem).wait()
                pltpu.make_async_copy(dst_hbm.at[wid, blk + 1],
                                      dst_v.at[1 - p], zsem).wait()

            @pl.when(c + 2 < NCHUNK)
            def _():
                nxt = o >= BCHUNK - 2
                p2 = lax.rem(blk + nxt.astype(jnp.int32), 2)
                o2 = jnp.where(nxt, o + 2 - BCHUNK, o + 2)
                gstart(p2, o2, lax.rem(c + 2, 3))

            return c_

        lax.fori_loop(0, BCHUNK, body, 0)
        return carry

    lax.fori_loop(0, NBLK, blk_body, 0)
    swait((NBLK - 1) % 2, 0, (NCHUNK - 1) % 3)
    plsc.subcore_barrier()
    pltpu.sync_copy(acc.at[pl.ds(base, STRIPE)],
                    out_hbm.at[cid, pl.ds(base, STRIPE)])


def _tc_norms_body(deg_ref, h_ref, xs_ref, nout_ref, nin_ref):
    out_deg = deg_ref[0, 0] + deg_ref[1, 0]
    in_deg = deg_ref[0, 1] + deg_ref[1, 1]
    norm_out = lax.rsqrt(jnp.maximum(out_deg, 1.0))[:N]
    norm_in = lax.rsqrt(jnp.maximum(in_deg, 1.0))[:N]
    xs_ref[...] = h_ref[...] * norm_out
    nout_ref[...] = norm_out
    nin_ref[...] = norm_in


def _tc_layer1_body(aggp_ref, nin_ref, nout_ref, w_ref, b_ref,
                    xs2_ref, skip_ref):
    agg = (aggp_ref[0] + aggp_ref[1])[:N] * nin_ref[...]
    x = jnp.dot(agg, w_ref[...], preferred_element_type=jnp.float32)
    x = jnp.maximum(x + b_ref[...], 0.0)
    skip_ref[...] = jnp.sum(x, axis=0, keepdims=True) * (1.0 / N)
    xs2_ref[...] = x * nout_ref[...]


def _tc_layer2_body(aggp_ref, nin_ref, w_ref, b_ref, skip1_ref, out_ref):
    agg = (aggp_ref[0] + aggp_ref[1])[:N] * nin_ref[...]
    x = jnp.dot(agg, w_ref[...], preferred_element_type=jnp.float32)
    x = jnp.maximum(x + b_ref[...], 0.0)
    out_ref[...] = skip1_ref[...] + 2.0 * (jnp.sum(x, axis=0, keepdims=True)
                                           * (1.0 / N))


_tc_norms = pl.pallas_call(
    _tc_norms_body,
    out_shape=(
        jax.ShapeDtypeStruct((N, D), jnp.float32),
        jax.ShapeDtypeStruct((N, 1), jnp.float32),
        jax.ShapeDtypeStruct((N, 1), jnp.float32),
    ),
)

_tc_layer1 = pl.pallas_call(
    _tc_layer1_body,
    out_shape=(
        jax.ShapeDtypeStruct((N, D), jnp.float32),
        jax.ShapeDtypeStruct((1, D), jnp.float32),
    ),
)

_tc_layer2 = pl.pallas_call(
    _tc_layer2_body,
    out_shape=jax.ShapeDtypeStruct((1, D), jnp.float32),
)


@jax.jit
def kernel(h, edge_index, W1, b1, W2, b2):
    src3 = edge_index[0].reshape(NW, NCHUNK, CHUNK)
    dst3 = edge_index[1].reshape(NW, NCHUNK, CHUNK)
    src4 = src3.reshape(NW, NBLK, BCHUNK, CHUNK)
    dst4 = dst3.reshape(NW, NBLK, BCHUNK, CHUNK)
    ones = jnp.ones((CHUNK,), jnp.float32)
    zeros1 = jnp.zeros((STRIPE,), jnp.float32)
    zeros2 = jnp.zeros((ZROWS, D), jnp.float32)

    deg = _sc_degrees(src3, dst3, ones, zeros1)
    deg4 = deg.reshape(NC, 2, NPAD, 1)
    xs1, norm_out, norm_in = _tc_norms(deg4, h)

    agg1 = _sc_scatter_rows(src4, dst4, xs1, zeros2)
    xs2, skip1 = _tc_layer1(agg1, norm_in, norm_out, W1, b1.reshape(1, D))

    agg2 = _sc_scatter_rows(src4, dst4, xs2, zeros2)
    return _tc_layer2(agg2, norm_in, W2, b2.reshape(1, D), skip1)


# speedup vs baseline: 1.4490x; 1.0104x over previous
import functools

import jax
import jax.numpy as jnp
from jax import lax
from jax.experimental import pallas as pl
from jax.experimental.pallas import tpu as pltpu
from jax.experimental.pallas import tpu_sc as plsc

N = 10000
E = 320000
D = 128

NC = 2
NS = 16
NW = NC * NS
EPW = E // NW
CHUNK = 80
NCHUNK = EPW // CHUNK
NBLK = 5
BCHUNK = NCHUNK // NBLK
NPAD = 10240
STRIPE = NPAD // NS
ZROWS = STRIPE // 4

_MESH = plsc.VectorSubcoreMesh(core_axis_name="c", subcore_axis_name="s")


@functools.partial(
    pl.kernel,
    out_type=jax.ShapeDtypeStruct((NC, 2, NPAD), jnp.float32),
    mesh=_MESH,
    scratch_types=[
        pltpu.VMEM((NCHUNK, CHUNK), jnp.int32),
        pltpu.VMEM((NCHUNK, CHUNK), jnp.int32),
        pltpu.VMEM((CHUNK,), jnp.float32),
        pltpu.VMEM_SHARED((NPAD,), jnp.float32),
        pltpu.VMEM_SHARED((NPAD,), jnp.float32),
        pltpu.SemaphoreType.DMA((2,)),
    ],
)
def _sc_degrees(src_hbm, dst_hbm, ones_hbm, zeros1_hbm, out_hbm,
                src_v, dst_v, ones_v, acc_out, acc_in, sems):
    cid = lax.axis_index("c")
    sid = lax.axis_index("s")
    wid = sid * NC + cid
    base = sid * STRIPE
    pltpu.sync_copy(zeros1_hbm, acc_out.at[pl.ds(base, STRIPE)])
    pltpu.sync_copy(zeros1_hbm, acc_in.at[pl.ds(base, STRIPE)])
    pltpu.sync_copy(src_hbm.at[wid], src_v)
    pltpu.sync_copy(dst_hbm.at[wid], dst_v)
    pltpu.sync_copy(ones_hbm, ones_v)
    plsc.subcore_barrier()

    def body(j, carry):
        pltpu.async_copy(ones_v, acc_out.at[src_v.at[j]], sems.at[0],
                         add=True)
        pltpu.async_copy(ones_v, acc_in.at[dst_v.at[j]], sems.at[1],
                         add=True)

        @pl.when(j >= 4)
        def _():
            pltpu.make_async_copy(ones_v, acc_out.at[src_v.at[0]],
                                  sems.at[0]).wait()
            pltpu.make_async_copy(ones_v, acc_in.at[dst_v.at[0]],
                                  sems.at[1]).wait()

        return carry

    lax.fori_loop(0, NCHUNK, body, 0)

    def drain(i, carry):
        pltpu.make_async_copy(ones_v, acc_out.at[src_v.at[0]],
                              sems.at[0]).wait()
        pltpu.make_async_copy(ones_v, acc_in.at[dst_v.at[0]],
                              sems.at[1]).wait()
        return carry

    lax.fori_loop(0, 4, drain, 0)
    plsc.subcore_barrier()
    pltpu.sync_copy(acc_out.at[pl.ds(base, STRIPE)],
                    out_hbm.at[cid, 0, pl.ds(base, STRIPE)])
    pltpu.sync_copy(acc_in.at[pl.ds(base, STRIPE)],
                    out_hbm.at[cid, 1, pl.ds(base, STRIPE)])


@functools.partial(
    pl.kernel,
    out_type=jax.ShapeDtypeStruct((NC, NPAD, D), jnp.float32),
    mesh=_MESH,
    scratch_types=[
        pltpu.VMEM((2, BCHUNK, CHUNK), jnp.int32),
        pltpu.VMEM((2, BCHUNK, CHUNK), jnp.int32),
        pltpu.VMEM((3, CHUNK, D), jnp.float32),
        pltpu.VMEM_SHARED((NPAD, D), jnp.float32),
        pltpu.SemaphoreType.DMA((3,)),
        pltpu.SemaphoreType.DMA,
    ],
)
def _sc_scatter_rows(src_hbm, dst_hbm, xs_hbm, zeros2_hbm, out_hbm,
                     src_v, dst_v, rows_v, acc, sems, zsem):
    cid = lax.axis_index("c")
    sid = lax.axis_index("s")
    wid = sid * NC + cid
    base = sid * STRIPE

    def zbody(i, carry):
        pltpu.async_copy(zeros2_hbm, acc.at[pl.ds(base + i * ZROWS, ZROWS)],
                         zsem)
        return carry

    lax.fori_loop(0, 4, zbody, 0)
    pltpu.sync_copy(src_hbm.at[wid, 0], src_v.at[0])
    pltpu.sync_copy(dst_hbm.at[wid, 0], dst_v.at[0])

    def zdrain(i, carry):
        pltpu.make_async_copy(zeros2_hbm,
                              acc.at[pl.ds(base + i * ZROWS, ZROWS)],
                              zsem).wait()
        return carry

    lax.fori_loop(0, 4, zdrain, 0)
    plsc.subcore_barrier()

    def gstart(p, o, b):
        pltpu.async_copy(xs_hbm.at[src_v.at[p, o]], rows_v.at[b],
                         sems.at[b])

    def gwait(p, o, b):
        pltpu.make_async_copy(xs_hbm.at[src_v.at[p, o]],
                              rows_v.at[b], sems.at[b]).wait()

    def sstart(p, o, b):
        pltpu.async_copy(rows_v.at[b], acc.at[dst_v.at[p, o]], sems.at[b],
                         add=True)

    def swait(p, o, b):
        pltpu.make_async_copy(rows_v.at[b], acc.at[dst_v.at[p, o]],
                              sems.at[b]).wait()

    gstart(0, 0, 0)
    gstart(0, 1, 1)

    def blk_body(blk, carry):
        p = lax.rem(blk, 2)

        def body(o, c_):
            c = blk * BCHUNK + o
            b = lax.rem(c, 3)

            gwait(p, o, b)
            sstart(p, o, b)

            @pl.when(c >= 1)
            def _():
                swait(p, 0, lax.rem(c + 2, 3))

            @pl.when(jnp.logical_and(o == 0, blk + 1 < NBLK))
            def _():
                pltpu.async_copy(src_hbm.at[wid, blk + 1],
                                 src_v.at[1 - p], zsem)
                pltpu.async_copy(dst_hbm.at[wid, blk + 1],
                                 dst_v.at[1 - p], zsem)

            @pl.when(jnp.logical_and(o == BCHUNK - 2, blk + 1 < NBLK))
            def _():
                pltpu.make_async_copy(src_hbm.at[wid, blk + 1],
                                      src_v.at[1 - p], zsem).wait()
                pltpu.make_async_copy(dst_hbm.at[wid, blk + 1],
                                      dst_v.at[1 - p], zsem).wait()

            @pl.when(c + 2 < NCHUNK)
            def _():
                nxt = o >= BCHUNK - 2
                p2 = lax.rem(blk + nxt.astype(jnp.int32), 2)
                o2 = jnp.where(nxt, o + 2 - BCHUNK, o + 2)
                gstart(p2, o2, lax.rem(c + 2, 3))

            return c_

        lax.fori_loop(0, BCHUNK, body, 0)
        return carry

    lax.fori_loop(0, NBLK, blk_body, 0)
    swait((NBLK - 1) % 2, 0, (NCHUNK - 1) % 3)
    plsc.subcore_barrier()
    pltpu.sync_copy(acc.at[pl.ds(base, STRIPE)],
                    out_hbm.at[cid, pl.ds(base, STRIPE)])


def _tc_norms_body(deg_ref, h_ref, xs_ref, nout_ref, nin_ref):
    out_deg = deg_ref[0, 0] + deg_ref[1, 0]
    in_deg = deg_ref[0, 1] + deg_ref[1, 1]
    norm_out = lax.rsqrt(jnp.maximum(out_deg, 1.0))[:N]
    norm_in = lax.rsqrt(jnp.maximum(in_deg, 1.0))[:N]
    xs_ref[...] = h_ref[...] * norm_out
    nout_ref[...] = norm_out
    nin_ref[...] = norm_in


def _tc_layer1_body(aggp_ref, nin_ref, nout_ref, w_ref, b_ref,
                    xs2_ref, skip_ref):
    agg = (aggp_ref[0] + aggp_ref[1])[:N] * nin_ref[...]
    x = jnp.dot(agg, w_ref[...], preferred_element_type=jnp.float32)
    x = jnp.maximum(x + b_ref[...], 0.0)
    skip_ref[...] = jnp.sum(x, axis=0, keepdims=True) * (1.0 / N)
    xs2_ref[...] = x * nout_ref[...]


def _tc_layer2_body(aggp_ref, nin_ref, w_ref, b_ref, skip1_ref, out_ref):
    agg = (aggp_ref[0] + aggp_ref[1])[:N] * nin_ref[...]
    x = jnp.dot(agg, w_ref[...], preferred_element_type=jnp.float32)
    x = jnp.maximum(x + b_ref[...], 0.0)
    out_ref[...] = skip1_ref[...] + 2.0 * (jnp.sum(x, axis=0, keepdims=True)
                                           * (1.0 / N))


_tc_norms = pl.pallas_call(
    _tc_norms_body,
    out_shape=(
        jax.ShapeDtypeStruct((N, D), jnp.float32),
        jax.ShapeDtypeStruct((N, 1), jnp.float32),
        jax.ShapeDtypeStruct((N, 1), jnp.float32),
    ),
)

_tc_layer1 = pl.pallas_call(
    _tc_layer1_body,
    out_shape=(
        jax.ShapeDtypeStruct((N, D), jnp.float32),
        jax.ShapeDtypeStruct((1, D), jnp.float32),
    ),
)

_tc_layer2 = pl.pallas_call(
    _tc_layer2_body,
    out_shape=jax.ShapeDtypeStruct((1, D), jnp.float32),
)


@jax.jit
def kernel(h, edge_index, W1, b1, W2, b2):
    src3 = edge_index[0].reshape(NW, NCHUNK, CHUNK)
    dst3 = edge_index[1].reshape(NW, NCHUNK, CHUNK)
    src4 = src3.reshape(NW, NBLK, BCHUNK, CHUNK)
    dst4 = dst3.reshape(NW, NBLK, BCHUNK, CHUNK)
    ones = jnp.ones((CHUNK,), jnp.float32)
    zeros1 = jnp.zeros((STRIPE,), jnp.float32)
    zeros2 = jnp.zeros((ZROWS, D), jnp.float32)

    deg = _sc_degrees(src3, dst3, ones, zeros1)
    deg4 = deg.reshape(NC, 2, NPAD, 1)
    xs1, norm_out, norm_in = _tc_norms(deg4, h)

    agg1 = _sc_scatter_rows(src4, dst4, xs1, zeros2)
    xs2, skip1 = _tc_layer1(agg1, norm_in, norm_out, W1, b1.reshape(1, D))

    agg2 = _sc_scatter_rows(src4, dst4, xs2, zeros2)
    return _tc_layer2(agg2, norm_in, W2, b2.reshape(1, D), skip1)
